# Initial kernel scaffold; baseline (speedup 1.0000x reference)
import jax
import jax.numpy as jnp
from jax.experimental import pallas as pl


def kernel(inputs, sampling_index):
    raise NotImplementedError("write your pallas kernel here")


# speedup vs baseline: 1.5504x; 1.5504x over previous
import functools

import jax
import jax.numpy as jnp
from jax import lax
from jax.experimental import pallas as pl
from jax.experimental.pallas import tpu as pltpu
from jax.experimental.pallas import tpu_sc as plsc

_B, _N, _C, _M = 4, 100000, 128, 25000
_ROWS = 128
_NCH = (_M + _ROWS - 1) // _ROWS
_TAIL = _M - (_NCH - 1) * _ROWS
_TASKS = _B * _NCH
_NW = 32
_ITERS = (_TASKS + _NW - 1) // _NW


def _sc_gather(table, idx2d):
    mesh = plsc.VectorSubcoreMesh(core_axis_name="c", subcore_axis_name="s")

    @functools.partial(
        pl.kernel,
        mesh=mesh,
        out_type=jax.ShapeDtypeStruct((_B * _M, _C), jnp.float32),
        scratch_types=[
            pltpu.VMEM((_ROWS,), jnp.int32),
            pltpu.VMEM((_ROWS, _C), jnp.float32),
            pltpu.SemaphoreType.DMA,
        ],
    )
    def k(table_hbm, idx_hbm, out_hbm, idx_v, rows_v, sem):
        wid = lax.axis_index("s") * 2 + lax.axis_index("c")

        def body(i, carry):
            t = wid + i * _NW

            @pl.when(t < _TASKS)
            def _():
                b = t // _NCH
                ch = t % _NCH
                pltpu.sync_copy(idx_hbm.at[ch], idx_v)
                off = b * _N
                for j in range(_ROWS // 16):
                    sl = pl.ds(j * 16, 16)
                    idx_v[sl] = idx_v[sl] + off
                pltpu.async_copy(table_hbm.at[idx_v], rows_v, sem).wait()
                obase = b * _M + ch * _ROWS

                @pl.when(ch < _NCH - 1)
                def _():
                    pltpu.sync_copy(rows_v, out_hbm.at[pl.ds(obase, _ROWS)])

                @pl.when(ch == _NCH - 1)
                def _():
                    pltpu.sync_copy(rows_v.at[pl.ds(0, _TAIL)],
                                    out_hbm.at[pl.ds(obase, _TAIL)])

            return carry

        lax.fori_loop(0, _ITERS, body, 0)

    return k(table, idx2d)


def kernel(inputs, sampling_index):
    table = inputs.reshape(_B * _N, _C)
    idx = sampling_index.astype(jnp.int32)
    idx_pad = jnp.zeros((_NCH * _ROWS,), jnp.int32).at[:_M].set(idx)
    idx2d = idx_pad.reshape(_NCH, _ROWS)
    out = _sc_gather(table, idx2d)
    return out.reshape(_B, _M, _C)


# speedup vs baseline: 2.0218x; 1.3041x over previous
import functools

import jax
import jax.numpy as jnp
from jax import lax
from jax.experimental import pallas as pl
from jax.experimental.pallas import tpu as pltpu
from jax.experimental.pallas import tpu_sc as plsc

_B, _N, _C, _M = 4, 100000, 128, 25000
_RPS = 128
_S = 2
_ROWS = _RPS * _S
_NCH = (_M + _ROWS - 1) // _ROWS
_TAIL = _M - (_NCH - 1) * _ROWS
_TASKS = _B * _NCH
_NW = 32
_ITERS = (_TASKS + _NW - 1) // _NW
_NBUF = 2


def _sc_gather(table, idx2d):
    mesh = plsc.VectorSubcoreMesh(core_axis_name="c", subcore_axis_name="s")

    @functools.partial(
        pl.kernel,
        mesh=mesh,
        out_type=jax.ShapeDtypeStruct((_B * _M, _C), jnp.float32),
        scratch_types=[
            pltpu.VMEM((_NBUF, _ROWS), jnp.int32),
            pltpu.VMEM((_NBUF, _ROWS, _C), jnp.float32),
            pltpu.SemaphoreType.DMA,
            pltpu.SemaphoreType.DMA,
        ],
    )
    def k(table_hbm, idx_hbm, out_hbm, idx_v, rows_v, sem0, sem1):
        wid = lax.axis_index("s") * 2 + lax.axis_index("c")
        sems = (sem0, sem1)

        def start_task(t, ib):
            b = t // _NCH
            ch = t % _NCH
            pltpu.sync_copy(idx_hbm.at[ch], idx_v.at[ib])
            off = b * _N
            for j in range(_ROWS // 16):
                sl = pl.ds(j * 16, 16)
                idx_v.at[ib][sl] = idx_v.at[ib][sl] + off
            for s in range(_S):
                pltpu.make_async_copy(
                    table_hbm.at[idx_v.at[ib].at[pl.ds(s * _RPS, _RPS)]],
                    rows_v.at[ib].at[pl.ds(s * _RPS, _RPS)],
                    sems[ib],
                ).start()

        def finish_task(t, ib):
            b = t // _NCH
            ch = t % _NCH
            for s in range(_S):
                pltpu.make_async_copy(
                    table_hbm.at[idx_v.at[ib].at[pl.ds(s * _RPS, _RPS)]],
                    rows_v.at[ib].at[pl.ds(s * _RPS, _RPS)],
                    sems[ib],
                ).wait()
            obase = b * _M + ch * _ROWS

            @pl.when(ch < _NCH - 1)
            def _():
                pltpu.sync_copy(rows_v.at[ib], out_hbm.at[pl.ds(obase, _ROWS)])

            @pl.when(ch == _NCH - 1)
            def _():
                pltpu.sync_copy(rows_v.at[ib].at[pl.ds(0, _TAIL)],
                                out_hbm.at[pl.ds(obase, _TAIL)])

        start_task(wid, 0)

        def body(i, carry):
            for p in range(_NBUF):
                kk = _NBUF * i + p
                t_next = wid + (kk + 1) * _NW

                @pl.when(t_next < _TASKS)
                def _():
                    start_task(t_next, (p + 1) % _NBUF)

                t_fin = wid + kk * _NW

                @pl.when(t_fin < _TASKS)
                def _():
                    finish_task(t_fin, p)

            return carry

        lax.fori_loop(0, (_ITERS + _NBUF - 1) // _NBUF, body, 0)

    return k(table, idx2d)


def kernel(inputs, sampling_index):
    table = inputs.reshape(_B * _N, _C)
    idx = sampling_index.astype(jnp.int32)
    idx_pad = jnp.zeros((_NCH * _ROWS,), jnp.int32).at[:_M].set(idx)
    idx2d = idx_pad.reshape(_NCH, _ROWS)
    out = _sc_gather(table, idx2d)
    return out.reshape(_B, _M, _C)


# speedup vs baseline: 2.1038x; 1.0406x over previous
import functools

import jax
import jax.numpy as jnp
from jax import lax
from jax.experimental import pallas as pl
from jax.experimental.pallas import tpu as pltpu
from jax.experimental.pallas import tpu_sc as plsc

_B, _N, _C, _M = 4, 100000, 128, 25000
_RPS = 128
_S = 2
_ROWS = _RPS * _S
_NCH = (_M + _ROWS - 1) // _ROWS
_TAIL = _M - (_NCH - 1) * _ROWS
_TASKS = _B * _NCH
_NW = 32
_KMAX = (_TASKS + _NW - 1) // _NW
_NBUF = 3


def _sc_gather(table, idx2d):
    mesh = plsc.VectorSubcoreMesh(core_axis_name="c", subcore_axis_name="s")

    @functools.partial(
        pl.kernel,
        mesh=mesh,
        out_type=jax.ShapeDtypeStruct((_B * _M, _C), jnp.float32),
        scratch_types=[
            pltpu.VMEM((_KMAX * _ROWS,), jnp.int32),
            pltpu.VMEM((_NBUF, _ROWS, _C), jnp.float32),
            pltpu.SemaphoreType.DMA,
            pltpu.SemaphoreType.DMA,
            pltpu.SemaphoreType.DMA,
            pltpu.SemaphoreType.DMA,
            pltpu.SemaphoreType.DMA,
            pltpu.SemaphoreType.DMA,
            pltpu.SemaphoreType.DMA,
        ],
    )
    def k(table_hbm, idx_hbm, out_hbm, idx_v, rows_v,
          sem_i, g0, g1, g2, w0, w1, w2):
        wid = lax.axis_index("s") * 2 + lax.axis_index("c")
        sem_g = (g0, g1, g2)
        sem_w = (w0, w1, w2)

        def task(kk):
            return wid + kk * _NW

        def task_parts(t):
            return t // _NCH, t % _NCH

        def idx_copy(kk, t):
            _, ch = task_parts(t)
            return pltpu.make_async_copy(
                idx_hbm.at[ch], idx_v.at[pl.ds(kk * _ROWS, _ROWS)], sem_i)

        def gather_copy(kk, s):
            ib = kk % _NBUF
            return pltpu.make_async_copy(
                table_hbm.at[idx_v.at[pl.ds(kk * _ROWS + s * _RPS, _RPS)]],
                rows_v.at[ib].at[pl.ds(s * _RPS, _RPS)],
                sem_g[ib],
            )

        def do_write(t, ib, start):
            b, ch = task_parts(t)
            obase = b * _M + ch * _ROWS

            @pl.when(ch < _NCH - 1)
            def _():
                cp = pltpu.make_async_copy(
                    rows_v.at[ib], out_hbm.at[pl.ds(obase, _ROWS)], sem_w[ib])
                cp.start() if start else cp.wait()

            @pl.when(ch == _NCH - 1)
            def _():
                cp = pltpu.make_async_copy(
                    rows_v.at[ib].at[pl.ds(0, _TAIL)],
                    out_hbm.at[pl.ds(obase, _TAIL)], sem_w[ib])
                cp.start() if start else cp.wait()

        for kk in range(_KMAX):
            t = task(kk)

            @pl.when(t < _TASKS)
            def _(kk=kk, t=t):
                idx_copy(kk, t).start()

        for kk in range(_KMAX):
            t = task(kk)

            @pl.when(t < _TASKS)
            def _(kk=kk, t=t):
                idx_copy(kk, t).wait()

        for kk in range(_KMAX):
            t = task(kk)

            @pl.when(t < _TASKS)
            def _(kk=kk, t=t):
                b, _ch = task_parts(t)
                off = b * _N
                for j in range(_ROWS // 16):
                    sl = pl.ds(kk * _ROWS + j * 16, 16)
                    idx_v[sl] = idx_v[sl] + off

        for kk in range(_NBUF - 1):
            t = task(kk)

            @pl.when(t < _TASKS)
            def _(kk=kk):
                for s in range(_S):
                    gather_copy(kk, s).start()

        for kk in range(_KMAX):
            t = task(kk)
            tn = task(kk + _NBUF - 1)

            if kk >= 1:
                @pl.when(task(kk - 1) < _TASKS)
                def _(kk=kk):
                    do_write(task(kk - 1), (kk - 1) % _NBUF, start=False)

            if kk + _NBUF - 1 < _KMAX:
                @pl.when(tn < _TASKS)
                def _(kk=kk):
                    for s in range(_S):
                        gather_copy(kk + _NBUF - 1, s).start()

            @pl.when(t < _TASKS)
            def _(kk=kk, t=t):
                ib = kk % _NBUF
                for s in range(_S):
                    gather_copy(kk, s).wait()
                do_write(t, ib, start=True)

        @pl.when(task(_KMAX - 1) < _TASKS)
        def _():
            do_write(task(_KMAX - 1), (_KMAX - 1) % _NBUF, start=False)

    return k(table, idx2d)


def kernel(inputs, sampling_index):
    table = inputs.reshape(_B * _N, _C)
    idx = sampling_index.astype(jnp.int32)
    idx_pad = jnp.zeros((_NCH * _ROWS,), jnp.int32).at[:_M].set(idx)
    idx2d = idx_pad.reshape(_NCH, _ROWS)
    out = _sc_gather(table, idx2d)
    return out.reshape(_B, _M, _C)


# speedup vs baseline: 2.1269x; 1.0110x over previous
import functools

import jax
import jax.numpy as jnp
from jax import lax
from jax.experimental import pallas as pl
from jax.experimental.pallas import tpu as pltpu
from jax.experimental.pallas import tpu_sc as plsc

_B, _N, _C, _M = 4, 100000, 128, 25000
_RPS = 128
_S = 2
_ROWS = _RPS * _S
_NCH = (_M + _ROWS - 1) // _ROWS
_TAIL = _M - (_NCH - 1) * _ROWS
_TASKS = _B * _NCH
_NSUB = 16
_FA = 164
_KMAX = (max(_FA, _TASKS - _FA) + _NSUB - 1) // _NSUB
_NBUF = 3


def _sc_gather(table, idx2d):
    mesh = plsc.VectorSubcoreMesh(core_axis_name="c", subcore_axis_name="s")

    @functools.partial(
        pl.kernel,
        mesh=mesh,
        out_type=jax.ShapeDtypeStruct((_B * _M, _C), jnp.float32),
        scratch_types=[
            pltpu.VMEM((_KMAX * _ROWS,), jnp.int32),
            pltpu.VMEM((_NBUF, _ROWS, _C), jnp.float32),
            pltpu.SemaphoreType.DMA,
            pltpu.SemaphoreType.DMA,
            pltpu.SemaphoreType.DMA,
            pltpu.SemaphoreType.DMA,
            pltpu.SemaphoreType.DMA,
            pltpu.SemaphoreType.DMA,
            pltpu.SemaphoreType.DMA,
        ],
    )
    def k(table_hbm, idx_hbm, out_hbm, idx_v, rows_v,
          sem_i, g0, g1, g2, w0, w1, w2):
        cid = lax.axis_index("c")
        sid = lax.axis_index("s")
        base = cid * _FA
        limit = _FA + cid * (_TASKS - _FA)
        sem_g = (g0, g1, g2)
        sem_w = (w0, w1, w2)

        def task(kk):
            return base + sid + kk * _NSUB

        def task_parts(t):
            return t // _NCH, t % _NCH

        def idx_copy(kk, t):
            _, ch = task_parts(t)
            return pltpu.make_async_copy(
                idx_hbm.at[ch], idx_v.at[pl.ds(kk * _ROWS, _ROWS)], sem_i)

        def gather_copy(kk, s):
            ib = kk % _NBUF
            return pltpu.make_async_copy(
                table_hbm.at[idx_v.at[pl.ds(kk * _ROWS + s * _RPS, _RPS)]],
                rows_v.at[ib].at[pl.ds(s * _RPS, _RPS)],
                sem_g[ib],
            )

        def do_write(t, ib, start):
            b, ch = task_parts(t)
            obase = b * _M + ch * _ROWS

            @pl.when(ch < _NCH - 1)
            def _():
                cp = pltpu.make_async_copy(
                    rows_v.at[ib], out_hbm.at[pl.ds(obase, _ROWS)], sem_w[ib])
                cp.start() if start else cp.wait()

            @pl.when(ch == _NCH - 1)
            def _():
                cp = pltpu.make_async_copy(
                    rows_v.at[ib].at[pl.ds(0, _TAIL)],
                    out_hbm.at[pl.ds(obase, _TAIL)], sem_w[ib])
                cp.start() if start else cp.wait()

        for kk in range(_KMAX):
            t = task(kk)

            @pl.when(t < limit)
            def _(kk=kk, t=t):
                idx_copy(kk, t).start()

        for kk in range(_KMAX):
            t = task(kk)

            @pl.when(t < limit)
            def _(kk=kk, t=t):
                idx_copy(kk, t).wait()

        for kk in range(_KMAX):
            t = task(kk)

            @pl.when(t < limit)
            def _(kk=kk, t=t):
                b, _ch = task_parts(t)
                off = b * _N
                for j in range(_ROWS // 16):
                    sl = pl.ds(kk * _ROWS + j * 16, 16)
                    idx_v[sl] = idx_v[sl] + off

        for kk in range(_NBUF - 1):
            t = task(kk)

            @pl.when(t < limit)
            def _(kk=kk):
                for s in range(_S):
                    gather_copy(kk, s).start()

        for kk in range(_KMAX):
            t = task(kk)
            tn = task(kk + _NBUF - 1)

            if kk >= 1:
                @pl.when(task(kk - 1) < limit)
                def _(kk=kk):
                    do_write(task(kk - 1), (kk - 1) % _NBUF, start=False)

            if kk + _NBUF - 1 < _KMAX:
                @pl.when(tn < limit)
                def _(kk=kk):
                    for s in range(_S):
                        gather_copy(kk + _NBUF - 1, s).start()

            @pl.when(t < limit)
            def _(kk=kk, t=t):
                ib = kk % _NBUF
                for s in range(_S):
                    gather_copy(kk, s).wait()
                do_write(t, ib, start=True)

        @pl.when(task(_KMAX - 1) < limit)
        def _():
            do_write(task(_KMAX - 1), (_KMAX - 1) % _NBUF, start=False)

    return k(table, idx2d)


def kernel(inputs, sampling_index):
    table = inputs.reshape(_B * _N, _C)
    idx = sampling_index.astype(jnp.int32)
    idx_pad = jnp.zeros((_NCH * _ROWS,), jnp.int32).at[:_M].set(idx)
    idx2d = idx_pad.reshape(_NCH, _ROWS)
    out = _sc_gather(table, idx2d)
    return out.reshape(_B, _M, _C)


# speedup vs baseline: 2.1330x; 1.0029x over previous
import functools

import jax
import jax.numpy as jnp
from jax import lax
from jax.experimental import pallas as pl
from jax.experimental.pallas import tpu as pltpu
from jax.experimental.pallas import tpu_sc as plsc

_B, _N, _C, _M = 4, 100000, 128, 25000
_RPS = 128
_S = 2
_ROWS = _RPS * _S
_NCH = (_M + _ROWS - 1) // _ROWS
_TAIL = _M - (_NCH - 1) * _ROWS
_TASKS = _B * _NCH
_NSUB = 16
_FA = 228
_KMAX = (max(_FA, _TASKS - _FA) + _NSUB - 1) // _NSUB
_NBUF = 3


def _sc_gather(table, idx2d):
    mesh = plsc.VectorSubcoreMesh(core_axis_name="c", subcore_axis_name="s")

    @functools.partial(
        pl.kernel,
        mesh=mesh,
        out_type=jax.ShapeDtypeStruct((_B * _M, _C), jnp.float32),
        scratch_types=[
            pltpu.VMEM((_KMAX * _ROWS,), jnp.int32),
            pltpu.VMEM((_NBUF, _ROWS, _C), jnp.float32),
            pltpu.SemaphoreType.DMA,
            pltpu.SemaphoreType.DMA,
            pltpu.SemaphoreType.DMA,
            pltpu.SemaphoreType.DMA,
            pltpu.SemaphoreType.DMA,
            pltpu.SemaphoreType.DMA,
            pltpu.SemaphoreType.DMA,
        ],
    )
    def k(table_hbm, idx_hbm, out_hbm, idx_v, rows_v,
          sem_i, g0, g1, g2, w0, w1, w2):
        cid = lax.axis_index("c")
        sid = lax.axis_index("s")
        base = cid * _FA
        limit = _FA + cid * (_TASKS - _FA)
        sem_g = (g0, g1, g2)
        sem_w = (w0, w1, w2)

        def task(kk):
            return base + sid + kk * _NSUB

        def task_parts(t):
            return t // _NCH, t % _NCH

        def idx_copy(kk, t):
            _, ch = task_parts(t)
            return pltpu.make_async_copy(
                idx_hbm.at[ch], idx_v.at[pl.ds(kk * _ROWS, _ROWS)], sem_i)

        def gather_copy(kk, s):
            ib = kk % _NBUF
            return pltpu.make_async_copy(
                table_hbm.at[idx_v.at[pl.ds(kk * _ROWS + s * _RPS, _RPS)]],
                rows_v.at[ib].at[pl.ds(s * _RPS, _RPS)],
                sem_g[ib],
            )

        def do_write(t, ib, start):
            b, ch = task_parts(t)
            obase = b * _M + ch * _ROWS

            @pl.when(ch < _NCH - 1)
            def _():
                cp = pltpu.make_async_copy(
                    rows_v.at[ib], out_hbm.at[pl.ds(obase, _ROWS)], sem_w[ib])
                cp.start() if start else cp.wait()

            @pl.when(ch == _NCH - 1)
            def _():
                cp = pltpu.make_async_copy(
                    rows_v.at[ib].at[pl.ds(0, _TAIL)],
                    out_hbm.at[pl.ds(obase, _TAIL)], sem_w[ib])
                cp.start() if start else cp.wait()

        for kk in range(_KMAX):
            t = task(kk)

            @pl.when(t < limit)
            def _(kk=kk, t=t):
                idx_copy(kk, t).start()

        for kk in range(_KMAX):
            t = task(kk)

            @pl.when(t < limit)
            def _(kk=kk, t=t):
                idx_copy(kk, t).wait()

        for kk in range(_KMAX):
            t = task(kk)

            @pl.when(t < limit)
            def _(kk=kk, t=t):
                b, _ch = task_parts(t)
                off = b * _N
                for j in range(_ROWS // 16):
                    sl = pl.ds(kk * _ROWS + j * 16, 16)
                    idx_v[sl] = idx_v[sl] + off

        for kk in range(_NBUF - 1):
            t = task(kk)

            @pl.when(t < limit)
            def _(kk=kk):
                for s in range(_S):
                    gather_copy(kk, s).start()

        for kk in range(_KMAX):
            t = task(kk)
            tn = task(kk + _NBUF - 1)

            if kk >= 1:
                @pl.when(task(kk - 1) < limit)
                def _(kk=kk):
                    do_write(task(kk - 1), (kk - 1) % _NBUF, start=False)

            if kk + _NBUF - 1 < _KMAX:
                @pl.when(tn < limit)
                def _(kk=kk):
                    for s in range(_S):
                        gather_copy(kk + _NBUF - 1, s).start()

            @pl.when(t < limit)
            def _(kk=kk, t=t):
                ib = kk % _NBUF
                for s in range(_S):
                    gather_copy(kk, s).wait()
                do_write(t, ib, start=True)

        @pl.when(task(_KMAX - 1) < limit)
        def _():
            do_write(task(_KMAX - 1), (_KMAX - 1) % _NBUF, start=False)

    return k(table, idx2d)


def kernel(inputs, sampling_index):
    table = inputs.reshape(_B * _N, _C)
    idx = sampling_index.astype(jnp.int32)
    idx_pad = jnp.zeros((_NCH * _ROWS,), jnp.int32).at[:_M].set(idx)
    idx2d = idx_pad.reshape(_NCH, _ROWS)
    out = _sc_gather(table, idx2d)
    return out.reshape(_B, _M, _C)


# speedup vs baseline: 2.6554x; 1.2449x over previous
import functools

import jax
import jax.numpy as jnp
from jax import lax
from jax.experimental import pallas as pl
from jax.experimental.pallas import tpu as pltpu
from jax.experimental.pallas import tpu_sc as plsc

_B, _N, _C, _M = 4, 100000, 128, 25000
_RPS = 128
_S = 2
_ROWS = _RPS * _S
_NCH = (_M + _ROWS - 1) // _ROWS
_TAIL = _M - (_NCH - 1) * _ROWS
_TAIL1 = _TAIL - _RPS
_TASKS = _B * _NCH
_NSUB = 16
_FA = _TASKS // 2
_KMAX = (max(_FA, _TASKS - _FA) + _NSUB - 1) // _NSUB
_NBUF = 3


def _sc_gather(table, idx):
    mesh = plsc.VectorSubcoreMesh(core_axis_name="c", subcore_axis_name="s")

    @functools.partial(
        pl.kernel,
        mesh=mesh,
        out_type=jax.ShapeDtypeStruct((_B * _M, _C), jnp.float32),
        scratch_types=[
            pltpu.VMEM((_KMAX * _ROWS,), jnp.int32),
            pltpu.VMEM((_NBUF, _ROWS, _C), jnp.float32),
            pltpu.SemaphoreType.DMA,
            pltpu.SemaphoreType.DMA,
            pltpu.SemaphoreType.DMA,
            pltpu.SemaphoreType.DMA,
            pltpu.SemaphoreType.DMA,
            pltpu.SemaphoreType.DMA,
            pltpu.SemaphoreType.DMA,
        ],
    )
    def k(table_hbm, idx_hbm, out_hbm, idx_v, rows_v,
          sem_i, g0, g1, g2, w0, w1, w2):
        cid = lax.axis_index("c")
        sid = lax.axis_index("s")
        base = cid * _FA
        limit = _FA + cid * (_TASKS - _FA)
        sem_g = (g0, g1, g2)
        sem_w = (w0, w1, w2)

        def task(kk):
            return base + sid + kk * _NSUB

        def task_parts(t):
            return t // _NCH, t % _NCH

        def do_idx(kk, t, start):
            _, ch = task_parts(t)

            @pl.when(ch < _NCH - 1)
            def _():
                cp = pltpu.make_async_copy(
                    idx_hbm.at[pl.ds(ch * _ROWS, _ROWS)],
                    idx_v.at[pl.ds(kk * _ROWS, _ROWS)], sem_i)
                cp.start() if start else cp.wait()

            @pl.when(ch == _NCH - 1)
            def _():
                cp = pltpu.make_async_copy(
                    idx_hbm.at[pl.ds((_NCH - 1) * _ROWS, _TAIL)],
                    idx_v.at[pl.ds(kk * _ROWS, _TAIL)], sem_i)
                cp.start() if start else cp.wait()

        def do_gathers(kk, t, start):
            ib = kk % _NBUF
            _, ch = task_parts(t)

            def one(s, nrows):
                cp = pltpu.make_async_copy(
                    table_hbm.at[idx_v.at[pl.ds(kk * _ROWS + s * _RPS, nrows)]],
                    rows_v.at[ib].at[pl.ds(s * _RPS, nrows)],
                    sem_g[ib])
                cp.start() if start else cp.wait()

            @pl.when(ch < _NCH - 1)
            def _():
                for s in range(_S):
                    one(s, _RPS)

            @pl.when(ch == _NCH - 1)
            def _():
                one(0, _RPS)
                one(1, _TAIL1)

        def do_write(t, ib, start):
            b, ch = task_parts(t)
            obase = b * _M + ch * _ROWS

            @pl.when(ch < _NCH - 1)
            def _():
                cp = pltpu.make_async_copy(
                    rows_v.at[ib], out_hbm.at[pl.ds(obase, _ROWS)], sem_w[ib])
                cp.start() if start else cp.wait()

            @pl.when(ch == _NCH - 1)
            def _():
                cp = pltpu.make_async_copy(
                    rows_v.at[ib].at[pl.ds(0, _TAIL)],
                    out_hbm.at[pl.ds(obase, _TAIL)], sem_w[ib])
                cp.start() if start else cp.wait()

        for kk in range(_KMAX):
            t = task(kk)

            @pl.when(t < limit)
            def _(kk=kk, t=t):
                do_idx(kk, t, start=True)

        for kk in range(_KMAX):
            t = task(kk)

            @pl.when(t < limit)
            def _(kk=kk, t=t):
                do_idx(kk, t, start=False)

        for kk in range(_KMAX):
            t = task(kk)

            @pl.when(t < limit)
            def _(kk=kk, t=t):
                b, _ch = task_parts(t)
                off = b * _N
                for j in range(_ROWS // 16):
                    sl = pl.ds(kk * _ROWS + j * 16, 16)
                    idx_v[sl] = idx_v[sl] + off

        for kk in range(_NBUF - 1):
            t = task(kk)

            @pl.when(t < limit)
            def _(kk=kk, t=t):
                do_gathers(kk, t, start=True)

        for kk in range(_KMAX):
            t = task(kk)

            if kk >= 1:
                @pl.when(task(kk - 1) < limit)
                def _(kk=kk):
                    do_write(task(kk - 1), (kk - 1) % _NBUF, start=False)

            if kk + _NBUF - 1 < _KMAX:
                tn = task(kk + _NBUF - 1)

                @pl.when(tn < limit)
                def _(kk=kk, tn=tn):
                    do_gathers(kk + _NBUF - 1, tn, start=True)

            @pl.when(t < limit)
            def _(kk=kk, t=t):
                do_gathers(kk, t, start=False)
                do_write(t, kk % _NBUF, start=True)

        @pl.when(task(_KMAX - 1) < limit)
        def _():
            do_write(task(_KMAX - 1), (_KMAX - 1) % _NBUF, start=False)

    return k(table, idx)


def kernel(inputs, sampling_index):
    table = inputs.reshape(_B * _N, _C)
    idx = sampling_index.astype(jnp.int32)
    out = _sc_gather(table, idx)
    return out.reshape(_B, _M, _C)
